# baseline (speedup 1.0000x reference)
import functools

import jax
import jax.numpy as jnp
from jax import lax
from jax.experimental import pallas as pl
from jax.experimental.pallas import tpu as pltpu
from jax.experimental.pallas import tpu_sc as plsc

_EMBED = 256
_IMG = 224
_NC = 2
_NS = 16
_NW = _NC * _NS
_L = 16
_CHUNK = 32
_NBUF = 4


def _sc_body(n_tok, fx_hbm, fy_hbm, desc_hbm, pos_hbm, out_hbm,
             fx_v, fy_v, idx_v, rows_v, desc_v, sem_g, sem_d, sem_o):
  b_per_w = n_tok // _NW
  n_chunks = b_per_w // _CHUNK
  wid = lax.axis_index("s") * _NC + lax.axis_index("c")
  w_base = wid * b_per_w

  pltpu.sync_copy(fx_hbm.at[pl.ds(w_base, b_per_w)], fx_v)
  pltpu.sync_copy(fy_hbm.at[pl.ds(w_base, b_per_w)], fy_v)

  def compute_idx(i, carry):
    sl = pl.ds(i * _L, _L)
    xi = jnp.clip((fx_v[sl] * _IMG).astype(jnp.int32), 0, _IMG - 1)
    yi = jnp.clip((fy_v[sl] * _IMG).astype(jnp.int32), 0, _IMG - 1)
    idx_v[sl] = yi * _IMG + xi
    return carry

  lax.fori_loop(0, b_per_w // _L, compute_idx, 0)

  def start_in(c, b):
    pltpu.async_copy(pos_hbm.at[idx_v.at[pl.ds(c * _CHUNK, _CHUNK)]],
                     rows_v.at[b], sem_g.at[b])
    pltpu.async_copy(desc_hbm.at[pl.ds(w_base + c * _CHUNK, _CHUNK)],
                     desc_v.at[b], sem_d.at[b])

  def wait_in(b):
    pltpu.make_async_copy(desc_hbm.at[pl.ds(0, _CHUNK)],
                          rows_v.at[b], sem_g.at[b]).wait()
    pltpu.make_async_copy(desc_hbm.at[pl.ds(0, _CHUNK)],
                          desc_v.at[b], sem_d.at[b]).wait()

  def start_s(c, b):
    pltpu.async_copy(desc_v.at[b],
                     out_hbm.at[pl.ds(w_base + c * _CHUNK, _CHUNK)],
                     sem_o.at[b])

  def wait_s(b):
    pltpu.make_async_copy(desc_v.at[b],
                          out_hbm.at[pl.ds(0, _CHUNK)], sem_o.at[b]).wait()

  start_in(0, 0)
  start_in(1, 1)

  def group(g, carry):
    for j in range(_NBUF):
      b = j
      bnn = (j + 2) % _NBUF
      c = g * _NBUF + j

      @pl.when(c + 2 < n_chunks)
      def _():
        @pl.when(c >= 2)
        def _():
          wait_s(bnn)
        start_in(c + 2, bnn)

      wait_in(b)

      def add_row(r, carry2):
        for k in range(_EMBED // _L):
          sl = pl.ds(k * _L, _L)
          plsc.addupdate(desc_v.at[b, r, sl], rows_v[b, r, sl])
        return carry2

      lax.fori_loop(0, _CHUNK, add_row, 0)
      start_s(c, b)
    return carry

  lax.fori_loop(0, n_chunks // _NBUF, group, 0)

  for b in range(_NBUF):
    wait_s(b)


@jax.jit
def kernel(discriptors, flows_in, pos_2d):
  shape = discriptors.shape
  n_tok = shape[0] * shape[1]
  d = discriptors.reshape(n_tok, _EMBED)
  fx = flows_in[..., 0].reshape(n_tok)
  fy = flows_in[..., 1].reshape(n_tok)
  p = pos_2d.reshape(_IMG * _IMG, _EMBED)

  b_per_w = n_tok // _NW
  mesh = plsc.VectorSubcoreMesh(core_axis_name="c", subcore_axis_name="s")
  out = pl.kernel(
      functools.partial(_sc_body, n_tok),
      out_type=jax.ShapeDtypeStruct((n_tok, _EMBED), jnp.float32),
      mesh=mesh,
      scratch_types=[
          pltpu.VMEM((b_per_w,), jnp.float32),
          pltpu.VMEM((b_per_w,), jnp.float32),
          pltpu.VMEM((b_per_w,), jnp.int32),
          pltpu.VMEM((_NBUF, _CHUNK, _EMBED), jnp.float32),
          pltpu.VMEM((_NBUF, _CHUNK, _EMBED), jnp.float32),
          pltpu.SemaphoreType.DMA((_NBUF,)),
          pltpu.SemaphoreType.DMA((_NBUF,)),
          pltpu.SemaphoreType.DMA((_NBUF,)),
      ],
  )(fx, fy, d, p)
  return out.reshape(shape)


# speedup vs baseline: 1.6438x; 1.6438x over previous
import functools

import jax
import jax.numpy as jnp
from jax import lax
from jax.experimental import pallas as pl
from jax.experimental.pallas import tpu as pltpu
from jax.experimental.pallas import tpu_sc as plsc

_EMBED = 256
_IMG = 224
_NC = 2
_NS = 16
_NW = _NC * _NS
_L = 16
_CHUNK = 64
_NBUF = 3


def _sc_body(n_tok, fx_hbm, fy_hbm, desc_hbm, pos_hbm, out_hbm,
             fx_v, fy_v, idx_v, rows_v, desc_v, sem_g, sem_d, sem_o):
  b_per_w = n_tok // _NW
  n_chunks = b_per_w // _CHUNK
  wid = lax.axis_index("s") * _NC + lax.axis_index("c")
  w_base = wid * b_per_w

  pltpu.sync_copy(fx_hbm.at[pl.ds(w_base, b_per_w)], fx_v)
  pltpu.sync_copy(fy_hbm.at[pl.ds(w_base, b_per_w)], fy_v)

  def compute_idx(i, carry):
    sl = pl.ds(i * _L, _L)
    xi = jnp.clip((fx_v[sl] * _IMG).astype(jnp.int32), 0, _IMG - 1)
    yi = jnp.clip((fy_v[sl] * _IMG).astype(jnp.int32), 0, _IMG - 1)
    idx_v[sl] = yi * _IMG + xi
    return carry

  lax.fori_loop(0, b_per_w // _L, compute_idx, 0)

  def start_in(c, b):
    pltpu.async_copy(pos_hbm.at[idx_v.at[pl.ds(c * _CHUNK, _CHUNK)]],
                     rows_v.at[b], sem_g.at[b])
    pltpu.async_copy(desc_hbm.at[pl.ds(w_base + c * _CHUNK, _CHUNK)],
                     desc_v.at[b], sem_d.at[b])

  def wait_in(b):
    pltpu.make_async_copy(desc_hbm.at[pl.ds(0, _CHUNK)],
                          rows_v.at[b], sem_g.at[b]).wait()
    pltpu.make_async_copy(desc_hbm.at[pl.ds(0, _CHUNK)],
                          desc_v.at[b], sem_d.at[b]).wait()

  def start_s(c, b):
    pltpu.async_copy(desc_v.at[b],
                     out_hbm.at[pl.ds(w_base + c * _CHUNK, _CHUNK)],
                     sem_o.at[b])

  def wait_s(b):
    pltpu.make_async_copy(desc_v.at[b],
                          out_hbm.at[pl.ds(0, _CHUNK)], sem_o.at[b]).wait()

  start_in(0, 0)
  start_in(1, 1)

  def chunk_step(c, carry):
    b = lax.rem(c, _NBUF)
    bn = lax.rem(c + 2, _NBUF)

    @pl.when(c + 2 < n_chunks)
    def _():
      @pl.when(c + 2 >= _NBUF)
      def _():
        wait_s(bn)
      start_in(c + 2, bn)

    wait_in(b)

    def add_row(r, carry2):
      for k in range(_EMBED // _L):
        sl = pl.ds(k * _L, _L)
        plsc.addupdate(desc_v.at[b, r, sl], rows_v[b, r, sl])
      return carry2

    lax.fori_loop(0, _CHUNK, add_row, 0)
    start_s(c, b)
    return carry

  lax.fori_loop(0, n_chunks, chunk_step, 0)

  for b in range(_NBUF):
    wait_s(b)


@jax.jit
def kernel(discriptors, flows_in, pos_2d):
  shape = discriptors.shape
  n_tok = shape[0] * shape[1]
  d = discriptors.reshape(n_tok, _EMBED)
  fx = flows_in[..., 0].reshape(n_tok)
  fy = flows_in[..., 1].reshape(n_tok)
  p = pos_2d.reshape(_IMG * _IMG, _EMBED)

  b_per_w = n_tok // _NW
  mesh = plsc.VectorSubcoreMesh(core_axis_name="c", subcore_axis_name="s")
  out = pl.kernel(
      functools.partial(_sc_body, n_tok),
      out_type=jax.ShapeDtypeStruct((n_tok, _EMBED), jnp.float32),
      mesh=mesh,
      scratch_types=[
          pltpu.VMEM((b_per_w,), jnp.float32),
          pltpu.VMEM((b_per_w,), jnp.float32),
          pltpu.VMEM((b_per_w,), jnp.int32),
          pltpu.VMEM((_NBUF, _CHUNK, _EMBED), jnp.float32),
          pltpu.VMEM((_NBUF, _CHUNK, _EMBED), jnp.float32),
          pltpu.SemaphoreType.DMA((_NBUF,)),
          pltpu.SemaphoreType.DMA((_NBUF,)),
          pltpu.SemaphoreType.DMA((_NBUF,)),
      ],
  )(fx, fy, d, p)
  return out.reshape(shape)


# speedup vs baseline: 1.6454x; 1.0009x over previous
import functools

import jax
import jax.numpy as jnp
from jax import lax
from jax.experimental import pallas as pl
from jax.experimental.pallas import tpu as pltpu
from jax.experimental.pallas import tpu_sc as plsc

_EMBED = 256
_IMG = 224
_NC = 2
_NS = 16
_NW = _NC * _NS
_L = 16
_CHUNK = 32
_NBUF = 6
_LOOK = _NBUF - 2


def _sc_body(n_tok, fx_hbm, fy_hbm, desc_hbm, pos_hbm, out_hbm,
             fx_v, fy_v, idx_v, rows_v, desc_v, sem_g, sem_d, sem_o):
  b_per_w = n_tok // _NW
  n_chunks = b_per_w // _CHUNK
  wid = lax.axis_index("s") * _NC + lax.axis_index("c")
  w_base = wid * b_per_w

  pltpu.sync_copy(fx_hbm.at[pl.ds(w_base, b_per_w)], fx_v)
  pltpu.sync_copy(fy_hbm.at[pl.ds(w_base, b_per_w)], fy_v)

  def compute_idx(i, carry):
    sl = pl.ds(i * _L, _L)
    xi = jnp.clip((fx_v[sl] * _IMG).astype(jnp.int32), 0, _IMG - 1)
    yi = jnp.clip((fy_v[sl] * _IMG).astype(jnp.int32), 0, _IMG - 1)
    idx_v[sl] = yi * _IMG + xi
    return carry

  lax.fori_loop(0, b_per_w // _L, compute_idx, 0)

  def start_in(c, b):
    pltpu.async_copy(pos_hbm.at[idx_v.at[pl.ds(c * _CHUNK, _CHUNK)]],
                     rows_v.at[b], sem_g.at[b])
    pltpu.async_copy(desc_hbm.at[pl.ds(w_base + c * _CHUNK, _CHUNK)],
                     desc_v.at[b], sem_d.at[b])

  def wait_in(b):
    pltpu.make_async_copy(desc_hbm.at[pl.ds(0, _CHUNK)],
                          rows_v.at[b], sem_g.at[b]).wait()
    pltpu.make_async_copy(desc_hbm.at[pl.ds(0, _CHUNK)],
                          desc_v.at[b], sem_d.at[b]).wait()

  def start_s(c, b):
    pltpu.async_copy(desc_v.at[b],
                     out_hbm.at[pl.ds(w_base + c * _CHUNK, _CHUNK)],
                     sem_o.at[b])

  def wait_s(b):
    pltpu.make_async_copy(desc_v.at[b],
                          out_hbm.at[pl.ds(0, _CHUNK)], sem_o.at[b]).wait()

  for j in range(_LOOK):
    start_in(j, j)

  def chunk_step(c, carry):
    b = lax.rem(c, _NBUF)
    bn = lax.rem(c + _LOOK, _NBUF)

    @pl.when(c + _LOOK < n_chunks)
    def _():
      @pl.when(c + _LOOK >= _NBUF)
      def _():
        wait_s(bn)
      start_in(c + _LOOK, bn)

    wait_in(b)

    def add_row(r, carry2):
      for k in range(_EMBED // _L):
        sl = pl.ds(k * _L, _L)
        plsc.addupdate(desc_v.at[b, r, sl], rows_v[b, r, sl])
      return carry2

    lax.fori_loop(0, _CHUNK, add_row, 0)
    start_s(c, b)
    return carry

  lax.fori_loop(0, n_chunks, chunk_step, 0)

  for b in range(_NBUF):
    wait_s(b)


@jax.jit
def kernel(discriptors, flows_in, pos_2d):
  shape = discriptors.shape
  n_tok = shape[0] * shape[1]
  d = discriptors.reshape(n_tok, _EMBED)
  fx = flows_in[..., 0].reshape(n_tok)
  fy = flows_in[..., 1].reshape(n_tok)
  p = pos_2d.reshape(_IMG * _IMG, _EMBED)

  b_per_w = n_tok // _NW
  mesh = plsc.VectorSubcoreMesh(core_axis_name="c", subcore_axis_name="s")
  out = pl.kernel(
      functools.partial(_sc_body, n_tok),
      out_type=jax.ShapeDtypeStruct((n_tok, _EMBED), jnp.float32),
      mesh=mesh,
      scratch_types=[
          pltpu.VMEM((b_per_w,), jnp.float32),
          pltpu.VMEM((b_per_w,), jnp.float32),
          pltpu.VMEM((b_per_w,), jnp.int32),
          pltpu.VMEM((_NBUF, _CHUNK, _EMBED), jnp.float32),
          pltpu.VMEM((_NBUF, _CHUNK, _EMBED), jnp.float32),
          pltpu.SemaphoreType.DMA((_NBUF,)),
          pltpu.SemaphoreType.DMA((_NBUF,)),
          pltpu.SemaphoreType.DMA((_NBUF,)),
      ],
  )(fx, fy, d, p)
  return out.reshape(shape)
